# Initial kernel scaffold; baseline (speedup 1.0000x reference)
import jax
import jax.numpy as jnp
from jax.experimental import pallas as pl


def kernel(mv, sc, pos, edge_index, W_msg, b_msg, g_msg, be_msg, W_upd, b_upd, g_upd, be_upd):
    raise NotImplementedError("write your pallas kernel here")


# speedup vs baseline: 2.3276x; 2.3276x over previous
import functools

import jax
import jax.numpy as jnp
from jax import lax
from jax.experimental import pallas as pl
from jax.experimental.pallas import tpu as pltpu
from jax.experimental.pallas import tpu_sc as plsc

N = 10000
E = 320000
DIM = 128
MV16 = 16
STEPS = 4

NC = 2
NS = 16
L = 16
KV = DIM // L

CHUNK = 80
EPS = E // NS
NCHUNK = EPS // CHUNK
EPC = E // (NC * NS)
NCHUNK_CNT = EPC // CHUNK
NPAD = 10240
RPT = NPAD // NS
ZROWS = 128

@functools.lru_cache(maxsize=None)
def _mesh():
    return plsc.VectorSubcoreMesh(core_axis_name="c", subcore_axis_name="s",
                                  num_cores=NC, num_subcores=NS)


def _v_rsqrt(v):
    i = lax.bitcast_convert_type(v, jnp.int32)
    i = 1597463007 - lax.shift_right_arithmetic(i, 1)
    y = lax.bitcast_convert_type(i, jnp.float32)
    for _ in range(3):
        y = y * (1.5 - 0.5 * v * y * y)
    return y


def _v_allsum(v):
    lane = lax.iota(jnp.int32, L)
    for sh in (8, 4, 2, 1):
        v = v + v.at[lane ^ sh].get(mode="promise_in_bounds")
    return v


def _v_gelu(x):
    z = 0.7978845608028654 * (x + 0.044715 * x * x * x)
    t = 1.0 - 2.0 / (1.0 + jnp.exp(2.0 * z))
    return 0.5 * x * (1.0 + t)


def _zero_fill(buf, nrows):
    w = buf.shape[1]
    zero = jnp.zeros((L,), jnp.float32)

    def body(r, _):
        for k in range(w // L):
            buf[r, pl.ds(k * L, L)] = zero
        return 0

    lax.fori_loop(0, nrows, body, 0)


def _msg_half(row, col, u_hbm, v_hbm, out_hbm, s, idxr, idxc, bufu, bufv, zbuf,
              acc, sem):
    _zero_fill(zbuf, ZROWS)
    for i in range(RPT // ZROWS):
        pltpu.sync_copy(zbuf, acc.at[pl.ds(s * RPT + i * ZROWS, ZROWS)])
    plsc.subcore_barrier()

    def chunk_body(j, _):
        base = s * EPS + j * CHUNK
        pltpu.sync_copy(row.at[pl.ds(base, CHUNK)], idxr)
        pltpu.sync_copy(col.at[pl.ds(base, CHUNK)], idxc)
        pltpu.async_copy(u_hbm.at[idxr], bufu, sem).wait()
        pltpu.async_copy(v_hbm.at[idxc], bufv, sem).wait()

        def edge_body(e, _):
            ms = []
            for k in range(KV):
                x = bufu[e, pl.ds(k * L, L)] + bufv[e, pl.ds(k * L, L)]
                ms.append(_v_gelu(x))
            tot = ms[0]
            totsq = ms[0] * ms[0]
            for k in range(1, KV):
                tot = tot + ms[k]
                totsq = totsq + ms[k] * ms[k]
            mu_v = _v_allsum(tot) * (1.0 / DIM)
            meansq = _v_allsum(totsq) * (1.0 / DIM)
            var = meansq - mu_v * mu_v
            rstd = _v_rsqrt(var + 1e-5)
            for k in range(KV):
                bufu[e, pl.ds(k * L, L)] = (ms[k] - mu_v) * rstd
            return 0

        lax.fori_loop(0, CHUNK, edge_body, 0)
        pltpu.sync_copy(bufu, acc.at[idxc], add=True)
        return 0

    lax.fori_loop(0, NCHUNK, chunk_body, 0)
    plsc.subcore_barrier()

    pltpu.sync_copy(acc.at[pl.ds(s * RPT, RPT)], out_hbm.at[pl.ds(s * RPT, RPT)])


@functools.lru_cache(maxsize=None)
def _sc_message_kernel():
    @functools.partial(
        pl.kernel,
        out_type=(
            jax.ShapeDtypeStruct((NPAD, DIM), jnp.float32),
            jax.ShapeDtypeStruct((NPAD, DIM), jnp.float32),
        ),
        mesh=_mesh(),
        scratch_types=[
            pltpu.VMEM((CHUNK,), jnp.int32),
            pltpu.VMEM((CHUNK,), jnp.int32),
            pltpu.VMEM((CHUNK, DIM), jnp.float32),
            pltpu.VMEM((CHUNK, DIM), jnp.float32),
            pltpu.VMEM((ZROWS, DIM), jnp.float32),
            pltpu.VMEM_SHARED((NPAD, DIM), jnp.float32),
            pltpu.SemaphoreType.DMA,
        ],
    )
    def _sc_message_impl(row, col, u0, v0, u1, v1, s0_out, s1_out,
                         idxr, idxc, bufu, bufv, zbuf, acc, sem):
        c = lax.axis_index("c")
        s = lax.axis_index("s")

        @pl.when(c == 0)
        def _():
            _msg_half(row, col, u0, v0, s0_out, s, idxr, idxc, bufu, bufv, zbuf, acc, sem)

        @pl.when(c == 1)
        def _():
            _msg_half(row, col, u1, v1, s1_out, s, idxr, idxc, bufu, bufv, zbuf, acc, sem)

    return _sc_message_impl


def _sc_message(row, col, u0, v0, u1, v1):
    return _sc_message_kernel()(row, col, u0, v0, u1, v1)


def _cnt_half(col, out_hbm, s, wid, idxc, ones_v, zbuf, acc):
    _zero_fill(zbuf, ZROWS)
    for i in range(RPT // ZROWS):
        pltpu.sync_copy(zbuf, acc.at[pl.ds(s * RPT + i * ZROWS, ZROWS)])

    one = jnp.full((L,), 1.0, jnp.float32)

    def fill_ones(r, _):
        for k in range(DIM // L):
            ones_v[r, pl.ds(k * L, L)] = one
        return 0

    lax.fori_loop(0, CHUNK, fill_ones, 0)
    plsc.subcore_barrier()

    def chunk_body(j, _):
        base = wid * EPC + j * CHUNK
        pltpu.sync_copy(col.at[pl.ds(base, CHUNK)], idxc)
        pltpu.sync_copy(ones_v, acc.at[idxc], add=True)
        return 0

    lax.fori_loop(0, NCHUNK_CNT, chunk_body, 0)
    plsc.subcore_barrier()
    pltpu.sync_copy(acc.at[pl.ds(s * RPT, RPT)], out_hbm.at[pl.ds(s * RPT, RPT)])


@functools.lru_cache(maxsize=None)
def _sc_count_kernel():
    @functools.partial(
        pl.kernel,
        out_type=(
            jax.ShapeDtypeStruct((NPAD, DIM), jnp.float32),
            jax.ShapeDtypeStruct((NPAD, DIM), jnp.float32),
        ),
        mesh=_mesh(),
        scratch_types=[
            pltpu.VMEM((CHUNK,), jnp.int32),
            pltpu.VMEM((CHUNK, DIM), jnp.float32),
            pltpu.VMEM((ZROWS, DIM), jnp.float32),
            pltpu.VMEM_SHARED((NPAD, DIM), jnp.float32),
        ],
    )
    def _sc_count_impl(col, c0_out, c1_out, idxc, ones_v, zbuf, acc):
        c = lax.axis_index("c")
        s = lax.axis_index("s")

        @pl.when(c == 0)
        def _():
            _cnt_half(col, c0_out, s, s, idxc, ones_v, zbuf, acc)

        @pl.when(c == 1)
        def _():
            _cnt_half(col, c1_out, s, NS + s, idxc, ones_v, zbuf, acc)

    return _sc_count_impl


def _sc_count(col):
    return _sc_count_kernel()(col)


BN = 1000
GRID = N // BN


def _ln(x, g, b):
    mu = jnp.mean(x, axis=-1, keepdims=True)
    var = jnp.mean(x * x, axis=-1, keepdims=True) - mu * mu
    return (x - mu) * lax.rsqrt(var + 1e-5) * g + b


def _tc_init_body(mv_ref, sc_ref, pos_ref, wa_ref, wb_ref, wc_ref, bm_ref,
                  u0_ref, v0_ref, u1_ref, v1_ref):
    pw = jnp.dot(pos_ref[...], wc_ref[...], preferred_element_type=jnp.float32)
    bm = bm_ref[...]
    for h_ref, u_ref, v_ref in ((mv_ref, u0_ref, v0_ref), (sc_ref, u1_ref, v1_ref)):
        h = h_ref[...]
        u_ref[...] = jnp.dot(h, wa_ref[...], preferred_element_type=jnp.float32) + pw + bm
        v_ref[...] = jnp.dot(h, wb_ref[...], preferred_element_type=jnp.float32) - pw


def _tc_upd_body(make_uv, mv_ref, sc_ref, s0_ref, s1_ref, c0_ref, c1_ref,
                 pos_ref, gm_ref, bem_ref, wu_ref, bu_ref, gu_ref, beu_ref,
                 wa_ref, wb_ref, wc_ref, bm_ref, *out_refs):
    cnt = c0_ref[...][:, :1] + c1_ref[...][:, :1]
    inv = 1.0 / jnp.maximum(cnt, 1.0)
    present = jnp.where(cnt > 0, 1.0, 0.0)
    gm = gm_ref[...]
    bem = bem_ref[...]
    wu = wu_ref[...]
    bu = bu_ref[...]
    gu = gu_ref[...]
    beu = beu_ref[...]
    if make_uv:
        mvo_ref, sco_ref, u0_ref, v0_ref, u1_ref, v1_ref = out_refs
        pw = jnp.dot(pos_ref[...], wc_ref[...], preferred_element_type=jnp.float32)
        pairs = ((mv_ref, s0_ref, mvo_ref, u0_ref, v0_ref),
                 (sc_ref, s1_ref, sco_ref, u1_ref, v1_ref))
    else:
        mvo_ref, sco_ref = out_refs
        pairs = ((mv_ref, s0_ref, mvo_ref, None, None),
                 (sc_ref, s1_ref, sco_ref, None, None))
    for h_ref, s_ref, ho_ref, u_ref, v_ref in pairs:
        h = h_ref[...]
        agg = s_ref[...] * inv * gm + bem * present
        upd = jnp.dot(h, wu[:DIM], preferred_element_type=jnp.float32)
        upd = upd + jnp.dot(agg, wu[DIM:], preferred_element_type=jnp.float32) + bu
        hn = h + _ln(upd, gu, beu)
        ho_ref[...] = hn
        if make_uv:
            u_ref[...] = jnp.dot(hn, wa_ref[...], preferred_element_type=jnp.float32) + pw + bm_ref[...]
            v_ref[...] = jnp.dot(hn, wb_ref[...], preferred_element_type=jnp.float32) - pw


def _row_spec(rows, cols):
    return pl.BlockSpec((rows, cols), lambda i: (i, 0))


def _full_spec(rows, cols):
    return pl.BlockSpec((rows, cols), lambda i: (0, 0))


_NODE = _row_spec(BN, DIM)
_OUT_NODE = jax.ShapeDtypeStruct((N, DIM), jnp.float32)


def _tc_init(mv, sc, pos, wa, wb, wc, bm):
    return pl.pallas_call(
        _tc_init_body,
        grid=(GRID,),
        in_specs=[_NODE, _NODE, _row_spec(BN, MV16), _full_spec(DIM, DIM),
                  _full_spec(DIM, DIM), _full_spec(MV16, DIM), _full_spec(1, DIM)],
        out_specs=[_NODE] * 4,
        out_shape=[_OUT_NODE] * 4,
    )(mv, sc, pos, wa, wb, wc, bm)


def _tc_update(mv, sc, s0, s1, c0, c1, pos, gm, bem, wu, bu, gu, beu,
               wa, wb, wc, bm, make_uv):
    n_out = 6 if make_uv else 2
    return pl.pallas_call(
        functools.partial(_tc_upd_body, make_uv),
        grid=(GRID,),
        in_specs=[_NODE, _NODE, _NODE, _NODE, _row_spec(BN, DIM), _row_spec(BN, DIM),
                  _row_spec(BN, MV16), _full_spec(1, DIM), _full_spec(1, DIM),
                  _full_spec(2 * DIM, DIM), _full_spec(1, DIM), _full_spec(1, DIM),
                  _full_spec(1, DIM), _full_spec(DIM, DIM), _full_spec(DIM, DIM),
                  _full_spec(MV16, DIM), _full_spec(1, DIM)],
        out_specs=[_NODE] * n_out,
        out_shape=[_OUT_NODE] * n_out,
    )(mv, sc, s0, s1, c0, c1, pos, gm, bem, wu, bu, gu, beu, wa, wb, wc, bm)


def kernel(mv, sc, pos, edge_index, W_msg, b_msg, g_msg, be_msg,
           W_upd, b_upd, g_upd, be_upd):
    row = edge_index[0]
    col = edge_index[1]
    c0, c1 = _sc_count(col)

    def r1(x):
        return x.reshape(1, DIM)

    wa = [W_msg[t, :DIM] for t in range(STEPS)]
    wb = [W_msg[t, DIM:2 * DIM] for t in range(STEPS)]
    wc = [W_msg[t, 2 * DIM:] for t in range(STEPS)]

    u0, v0, u1, v1 = _tc_init(mv, sc, pos, wa[0], wb[0], wc[0], r1(b_msg[0]))
    for t in range(STEPS):
        s0, s1 = _sc_message(row, col, u0, v0, u1, v1)
        tn = min(t + 1, STEPS - 1)
        outs = _tc_update(
            mv, sc, s0, s1, c0, c1, pos, r1(g_msg[t]), r1(be_msg[t]),
            W_upd[t], r1(b_upd[t]), r1(g_upd[t]), r1(be_upd[t]),
            wa[tn], wb[tn], wc[tn], r1(b_msg[tn]), make_uv=(t + 1 < STEPS))
        if t + 1 < STEPS:
            mv, sc, u0, v0, u1, v1 = outs
        else:
            mv, sc = outs
    return mv, sc


# speedup vs baseline: 6.0611x; 2.6040x over previous
import functools

import jax
import jax.numpy as jnp
from jax import lax
from jax.experimental import pallas as pl
from jax.experimental.pallas import tpu as pltpu
from jax.experimental.pallas import tpu_sc as plsc

N = 10000
E = 320000
DIM = 128
MV16 = 16
STEPS = 4

NC = 2
NS = 16
L = 16
KV = DIM // L

CHUNK = 40
EPS = E // NS
NCHUNK = EPS // CHUNK
EPC = E // (NC * NS)
NCHUNK_CNT = EPC // CHUNK
NPAD = 10240
RPT = NPAD // NS
ZROWS = 64

@functools.lru_cache(maxsize=None)
def _mesh():
    return plsc.VectorSubcoreMesh(core_axis_name="c", subcore_axis_name="s",
                                  num_cores=NC, num_subcores=NS)


def _v_rsqrt(v):
    i = lax.bitcast_convert_type(v, jnp.int32)
    i = 1597463007 - lax.shift_right_arithmetic(i, 1)
    y = lax.bitcast_convert_type(i, jnp.float32)
    for _ in range(3):
        y = y * (1.5 - 0.5 * v * y * y)
    return y


def _v_allsum(v):
    lane = lax.iota(jnp.int32, L)
    for sh in (8, 4, 2, 1):
        v = v + v.at[lane ^ sh].get(mode="promise_in_bounds")
    return v


def _v_gelu(x):
    z = 0.7978845608028654 * (x + 0.044715 * x * x * x)
    t = 1.0 - 2.0 / (1.0 + jnp.exp(2.0 * z))
    return 0.5 * x * (1.0 + t)


def _zero_fill(buf, nrows):
    w = buf.shape[1]
    zero = jnp.zeros((L,), jnp.float32)

    def body(r, _):
        for k in range(w // L):
            buf[r, pl.ds(k * L, L)] = zero
        return 0

    lax.fori_loop(0, nrows, body, 0)


def _edge_chunk(bufu, bufv, bufm):
    def edge_body(e, _):
        ms = []
        for k in range(KV):
            x = bufu[e, pl.ds(k * L, L)] + bufv[e, pl.ds(k * L, L)]
            ms.append(_v_gelu(x))
        tot = ms[0]
        totsq = ms[0] * ms[0]
        for k in range(1, KV):
            tot = tot + ms[k]
            totsq = totsq + ms[k] * ms[k]
        mu_v = _v_allsum(tot) * (1.0 / DIM)
        meansq = _v_allsum(totsq) * (1.0 / DIM)
        var = meansq - mu_v * mu_v
        rstd = _v_rsqrt(var + 1e-5)
        for k in range(KV):
            bufm[e, pl.ds(k * L, L)] = (ms[k] - mu_v) * rstd
        return 0

    lax.fori_loop(0, CHUNK, edge_body, 0)


def _msg_half(row, col, u_hbm, v_hbm, out_hbm, s, idxrs, idxcs, idxss,
              bufus, bufvs, bufms, acc, semus, semvs, semss, semirs, semics):
    _zero_fill(bufms[0], CHUNK)
    for i in range(RPT // CHUNK):
        pltpu.sync_copy(bufms[0], acc.at[pl.ds(s * RPT + i * CHUNK, CHUNK)])

    def idx_issue(j, q):
        base = s * EPS + j * CHUNK
        pltpu.async_copy(row.at[pl.ds(base, CHUNK)], idxrs[q], semirs[q])
        pltpu.async_copy(col.at[pl.ds(base, CHUNK)], idxcs[q], semics[q])

    def idx_wait(j, q):
        base = s * EPS + j * CHUNK
        pltpu.make_async_copy(row.at[pl.ds(base, CHUNK)], idxrs[q], semirs[q]).wait()
        pltpu.make_async_copy(col.at[pl.ds(base, CHUNK)], idxcs[q], semics[q]).wait()

    def gather(q, b):
        pltpu.async_copy(u_hbm.at[idxrs[q]], bufus[b], semus[b])
        pltpu.async_copy(v_hbm.at[idxcs[q]], bufvs[b], semvs[b])

    def gather_wait(q, b):
        pltpu.make_async_copy(u_hbm.at[idxrs[q]], bufus[b], semus[b]).wait()
        pltpu.make_async_copy(v_hbm.at[idxcs[q]], bufvs[b], semvs[b]).wait()

    def scatter(b):
        pltpu.async_copy(bufms[b], acc.at[idxss[b]], semss[b], add=True)

    def scatter_wait(b):
        pltpu.make_async_copy(bufms[b], acc.at[idxss[b]], semss[b]).wait()

    for q in range(4):
        idx_issue(q, q)
    for b in range(2):
        idx_wait(b, b)
        gather(b, b)
    plsc.subcore_barrier()

    def loop_body(jj, _):
        for t in range(4):
            b = t % 2
            q = t
            j = jj * 4 + t
            gather_wait(q, b)

            @pl.when(j >= 2)
            def _():
                scatter_wait(b)

            _edge_chunk(bufus[b], bufvs[b], bufms[b])
            for o in (0, 16, CHUNK - L):
                idxss[b][pl.ds(o, L)] = idxcs[q][pl.ds(o, L)]
            scatter(b)

            @pl.when(j + 2 < NCHUNK)
            def _():
                idx_wait(j + 2, (t + 2) % 4)
                gather((t + 2) % 4, b)

            @pl.when(j + 4 < NCHUNK)
            def _():
                idx_issue(j + 4, q)
        return 0

    lax.fori_loop(0, NCHUNK // 4, loop_body, 0)
    for b in range(2):
        scatter_wait(b)
    plsc.subcore_barrier()

    pltpu.sync_copy(acc.at[pl.ds(s * RPT, RPT)], out_hbm.at[pl.ds(s * RPT, RPT)])


@functools.lru_cache(maxsize=None)
def _sc_message_kernel():
    @functools.partial(
        pl.kernel,
        out_type=(
            jax.ShapeDtypeStruct((NPAD, DIM), jnp.float32),
            jax.ShapeDtypeStruct((NPAD, DIM), jnp.float32),
        ),
        mesh=_mesh(),
        scratch_types=[
            [pltpu.VMEM((CHUNK,), jnp.int32)] * 4,
            [pltpu.VMEM((CHUNK,), jnp.int32)] * 4,
            [pltpu.VMEM((CHUNK,), jnp.int32)] * 2,
            [pltpu.VMEM((CHUNK, DIM), jnp.float32)] * 2,
            [pltpu.VMEM((CHUNK, DIM), jnp.float32)] * 2,
            [pltpu.VMEM((CHUNK, DIM), jnp.float32)] * 2,
            pltpu.VMEM_SHARED((NPAD, DIM), jnp.float32),
            [pltpu.SemaphoreType.DMA] * 2,
            [pltpu.SemaphoreType.DMA] * 2,
            [pltpu.SemaphoreType.DMA] * 2,
            [pltpu.SemaphoreType.DMA] * 4,
            [pltpu.SemaphoreType.DMA] * 4,
        ],
    )
    def _sc_message_impl(row, col, u0, v0, u1, v1, s0_out, s1_out,
                         idxrs, idxcs, idxss, bufus, bufvs, bufms, acc,
                         semus, semvs, semss, semirs, semics):
        c = lax.axis_index("c")
        s = lax.axis_index("s")

        @pl.when(c == 0)
        def _():
            _msg_half(row, col, u0, v0, s0_out, s, idxrs, idxcs, idxss,
                      bufus, bufvs, bufms, acc, semus, semvs, semss,
                      semirs, semics)

        @pl.when(c == 1)
        def _():
            _msg_half(row, col, u1, v1, s1_out, s, idxrs, idxcs, idxss,
                      bufus, bufvs, bufms, acc, semus, semvs, semss,
                      semirs, semics)

    return _sc_message_impl


def _sc_message(row, col, u0, v0, u1, v1):
    return _sc_message_kernel()(row, col, u0, v0, u1, v1)


def _cnt_half(col, out_hbm, s, wid, idxc, ones_v, zbuf, acc):
    _zero_fill(zbuf, ZROWS)
    for i in range(RPT // ZROWS):
        pltpu.sync_copy(zbuf, acc.at[pl.ds(s * RPT + i * ZROWS, ZROWS)])

    one = jnp.full((L,), 1.0, jnp.float32)

    def fill_ones(r, _):
        for k in range(DIM // L):
            ones_v[r, pl.ds(k * L, L)] = one
        return 0

    lax.fori_loop(0, CHUNK, fill_ones, 0)
    plsc.subcore_barrier()

    def chunk_body(j, _):
        base = wid * EPC + j * CHUNK
        pltpu.sync_copy(col.at[pl.ds(base, CHUNK)], idxc)
        pltpu.sync_copy(ones_v, acc.at[idxc], add=True)
        return 0

    lax.fori_loop(0, NCHUNK_CNT, chunk_body, 0)
    plsc.subcore_barrier()
    pltpu.sync_copy(acc.at[pl.ds(s * RPT, RPT)], out_hbm.at[pl.ds(s * RPT, RPT)])


@functools.lru_cache(maxsize=None)
def _sc_count_kernel():
    @functools.partial(
        pl.kernel,
        out_type=(
            jax.ShapeDtypeStruct((NPAD, DIM), jnp.float32),
            jax.ShapeDtypeStruct((NPAD, DIM), jnp.float32),
        ),
        mesh=_mesh(),
        scratch_types=[
            pltpu.VMEM((CHUNK,), jnp.int32),
            pltpu.VMEM((CHUNK, DIM), jnp.float32),
            pltpu.VMEM((ZROWS, DIM), jnp.float32),
            pltpu.VMEM_SHARED((NPAD, DIM), jnp.float32),
        ],
    )
    def _sc_count_impl(col, c0_out, c1_out, idxc, ones_v, zbuf, acc):
        c = lax.axis_index("c")
        s = lax.axis_index("s")

        @pl.when(c == 0)
        def _():
            _cnt_half(col, c0_out, s, s, idxc, ones_v, zbuf, acc)

        @pl.when(c == 1)
        def _():
            _cnt_half(col, c1_out, s, NS + s, idxc, ones_v, zbuf, acc)

    return _sc_count_impl


def _sc_count(col):
    return _sc_count_kernel()(col)


BN = 1000
GRID = N // BN


def _ln(x, g, b):
    mu = jnp.mean(x, axis=-1, keepdims=True)
    var = jnp.mean(x * x, axis=-1, keepdims=True) - mu * mu
    return (x - mu) * lax.rsqrt(var + 1e-5) * g + b


def _tc_init_body(mv_ref, sc_ref, pos_ref, wa_ref, wb_ref, wc_ref, bm_ref,
                  u0_ref, v0_ref, u1_ref, v1_ref):
    pw = jnp.dot(pos_ref[...], wc_ref[...], preferred_element_type=jnp.float32)
    bm = bm_ref[...]
    for h_ref, u_ref, v_ref in ((mv_ref, u0_ref, v0_ref), (sc_ref, u1_ref, v1_ref)):
        h = h_ref[...]
        u_ref[...] = jnp.dot(h, wa_ref[...], preferred_element_type=jnp.float32) + pw + bm
        v_ref[...] = jnp.dot(h, wb_ref[...], preferred_element_type=jnp.float32) - pw


def _tc_upd_body(make_uv, mv_ref, sc_ref, s0_ref, s1_ref, c0_ref, c1_ref,
                 pos_ref, gm_ref, bem_ref, wu_ref, bu_ref, gu_ref, beu_ref,
                 wa_ref, wb_ref, wc_ref, bm_ref, *out_refs):
    cnt = c0_ref[...][:, :1] + c1_ref[...][:, :1]
    inv = 1.0 / jnp.maximum(cnt, 1.0)
    present = jnp.where(cnt > 0, 1.0, 0.0)
    gm = gm_ref[...]
    bem = bem_ref[...]
    wu = wu_ref[...]
    bu = bu_ref[...]
    gu = gu_ref[...]
    beu = beu_ref[...]
    if make_uv:
        mvo_ref, sco_ref, u0_ref, v0_ref, u1_ref, v1_ref = out_refs
        pw = jnp.dot(pos_ref[...], wc_ref[...], preferred_element_type=jnp.float32)
        pairs = ((mv_ref, s0_ref, mvo_ref, u0_ref, v0_ref),
                 (sc_ref, s1_ref, sco_ref, u1_ref, v1_ref))
    else:
        mvo_ref, sco_ref = out_refs
        pairs = ((mv_ref, s0_ref, mvo_ref, None, None),
                 (sc_ref, s1_ref, sco_ref, None, None))
    for h_ref, s_ref, ho_ref, u_ref, v_ref in pairs:
        h = h_ref[...]
        agg = s_ref[...] * inv * gm + bem * present
        upd = jnp.dot(h, wu[:DIM], preferred_element_type=jnp.float32)
        upd = upd + jnp.dot(agg, wu[DIM:], preferred_element_type=jnp.float32) + bu
        hn = h + _ln(upd, gu, beu)
        ho_ref[...] = hn
        if make_uv:
            u_ref[...] = jnp.dot(hn, wa_ref[...], preferred_element_type=jnp.float32) + pw + bm_ref[...]
            v_ref[...] = jnp.dot(hn, wb_ref[...], preferred_element_type=jnp.float32) - pw


def _row_spec(rows, cols):
    return pl.BlockSpec((rows, cols), lambda i: (i, 0))


def _full_spec(rows, cols):
    return pl.BlockSpec((rows, cols), lambda i: (0, 0))


_NODE = _row_spec(BN, DIM)
_OUT_NODE = jax.ShapeDtypeStruct((N, DIM), jnp.float32)


def _tc_init(mv, sc, pos, wa, wb, wc, bm):
    return pl.pallas_call(
        _tc_init_body,
        grid=(GRID,),
        in_specs=[_NODE, _NODE, _row_spec(BN, MV16), _full_spec(DIM, DIM),
                  _full_spec(DIM, DIM), _full_spec(MV16, DIM), _full_spec(1, DIM)],
        out_specs=[_NODE] * 4,
        out_shape=[_OUT_NODE] * 4,
    )(mv, sc, pos, wa, wb, wc, bm)


def _tc_update(mv, sc, s0, s1, c0, c1, pos, gm, bem, wu, bu, gu, beu,
               wa, wb, wc, bm, make_uv):
    n_out = 6 if make_uv else 2
    return pl.pallas_call(
        functools.partial(_tc_upd_body, make_uv),
        grid=(GRID,),
        in_specs=[_NODE, _NODE, _NODE, _NODE, _row_spec(BN, DIM), _row_spec(BN, DIM),
                  _row_spec(BN, MV16), _full_spec(1, DIM), _full_spec(1, DIM),
                  _full_spec(2 * DIM, DIM), _full_spec(1, DIM), _full_spec(1, DIM),
                  _full_spec(1, DIM), _full_spec(DIM, DIM), _full_spec(DIM, DIM),
                  _full_spec(MV16, DIM), _full_spec(1, DIM)],
        out_specs=[_NODE] * n_out,
        out_shape=[_OUT_NODE] * n_out,
    )(mv, sc, s0, s1, c0, c1, pos, gm, bem, wu, bu, gu, beu, wa, wb, wc, bm)


def kernel(mv, sc, pos, edge_index, W_msg, b_msg, g_msg, be_msg,
           W_upd, b_upd, g_upd, be_upd):
    row = edge_index[0]
    col = edge_index[1]
    c0, c1 = _sc_count(col)

    def r1(x):
        return x.reshape(1, DIM)

    wa = [W_msg[t, :DIM] for t in range(STEPS)]
    wb = [W_msg[t, DIM:2 * DIM] for t in range(STEPS)]
    wc = [W_msg[t, 2 * DIM:] for t in range(STEPS)]

    u0, v0, u1, v1 = _tc_init(mv, sc, pos, wa[0], wb[0], wc[0], r1(b_msg[0]))
    for t in range(STEPS):
        s0, s1 = _sc_message(row, col, u0, v0, u1, v1)
        tn = min(t + 1, STEPS - 1)
        outs = _tc_update(
            mv, sc, s0, s1, c0, c1, pos, r1(g_msg[t]), r1(be_msg[t]),
            W_upd[t], r1(b_upd[t]), r1(g_upd[t]), r1(be_upd[t]),
            wa[tn], wb[tn], wc[tn], r1(b_msg[tn]), make_uv=(t + 1 < STEPS))
        if t + 1 < STEPS:
            mv, sc, u0, v0, u1, v1 = outs
        else:
            mv, sc = outs
    return mv, sc


# speedup vs baseline: 12.8307x; 2.1169x over previous
import functools

import jax
import jax.numpy as jnp
from jax import lax
from jax.experimental import pallas as pl
from jax.experimental.pallas import tpu as pltpu
from jax.experimental.pallas import tpu_sc as plsc

N = 10000
E = 320000
DIM = 128
MV16 = 16
STEPS = 4

NC = 2
NS = 16
L = 16
KV = DIM // L

CHUNK = 40
EPS = E // NS
NCHUNK = EPS // CHUNK
EPC = E // (NC * NS)
NCHUNK_CNT = EPC // CHUNK
NPAD = 10240
RPT = NPAD // NS
ZROWS = 64

@functools.lru_cache(maxsize=None)
def _mesh():
    return plsc.VectorSubcoreMesh(core_axis_name="c", subcore_axis_name="s",
                                  num_cores=NC, num_subcores=NS)


def _v_rsqrt(v):
    i = lax.bitcast_convert_type(v, jnp.int32)
    i = 1597463007 - lax.shift_right_arithmetic(i, 1)
    y = lax.bitcast_convert_type(i, jnp.float32)
    for _ in range(3):
        y = y * (1.5 - 0.5 * v * y * y)
    return y


_GC0 = 2.0 * 0.7978845608028654
_GC1 = _GC0 * 0.044715


def _v_gelu_fast(x):
    x2 = x * x
    p = _GC1 * x2 + _GC0
    e = jnp.exp(x * p)
    r = 1.0 / (1.0 + e)
    return x - x * r


def _v_allsum(v):
    lane = lax.iota(jnp.int32, L)
    for sh in (8, 4, 2, 1):
        v = v + v.at[lane ^ sh].get(mode="promise_in_bounds")
    return v


def _zero_fill(buf, nrows):
    w = buf.shape[1]
    zero = jnp.zeros((L,), jnp.float32)

    def body(r, _):
        for k in range(w // L):
            buf[r, pl.ds(k * L, L)] = zero
        return 0

    lax.fori_loop(0, nrows, body, 0)


def _edge_chunk(bufu, bufv, bufm):
    @functools.partial(plsc.parallel_loop, 0, CHUNK, unroll=2)
    def edge_body(e):
        ms = []
        for k in range(KV):
            x = bufu[e, pl.ds(k * L, L)] + bufv[e, pl.ds(k * L, L)]
            ms.append(_v_gelu_fast(x))
        tot = ms[0]
        totsq = ms[0] * ms[0]
        for k in range(1, KV):
            tot = tot + ms[k]
            totsq = totsq + ms[k] * ms[k]
        mu_v = _v_allsum(tot) * (1.0 / DIM)
        meansq = _v_allsum(totsq) * (1.0 / DIM)
        var = meansq - mu_v * mu_v
        rstd = _v_rsqrt(var + 1e-5)
        for k in range(KV):
            bufm[e, pl.ds(k * L, L)] = (ms[k] - mu_v) * rstd


def _msg_half(row, col, u_hbm, v_hbm, out_hbm, s, idxrs, idxcs, idxss,
              bufus, bufvs, bufms, acc, semus, semvs, semss, semirs, semics):
    _zero_fill(bufms[0], CHUNK)
    for i in range(RPT // CHUNK):
        pltpu.sync_copy(bufms[0], acc.at[pl.ds(s * RPT + i * CHUNK, CHUNK)])

    def idx_issue(j, q):
        base = s * EPS + j * CHUNK
        pltpu.async_copy(row.at[pl.ds(base, CHUNK)], idxrs[q], semirs[q])
        pltpu.async_copy(col.at[pl.ds(base, CHUNK)], idxcs[q], semics[q])

    def idx_wait(j, q):
        base = s * EPS + j * CHUNK
        pltpu.make_async_copy(row.at[pl.ds(base, CHUNK)], idxrs[q], semirs[q]).wait()
        pltpu.make_async_copy(col.at[pl.ds(base, CHUNK)], idxcs[q], semics[q]).wait()

    def gather(q, b):
        pltpu.async_copy(u_hbm.at[idxrs[q]], bufus[b], semus[b])
        pltpu.async_copy(v_hbm.at[idxcs[q]], bufvs[b], semvs[b])

    def gather_wait(q, b):
        pltpu.make_async_copy(u_hbm.at[idxrs[q]], bufus[b], semus[b]).wait()
        pltpu.make_async_copy(v_hbm.at[idxcs[q]], bufvs[b], semvs[b]).wait()

    def scatter(b):
        pltpu.async_copy(bufms[b], acc.at[idxss[b]], semss[b], add=True)

    def scatter_wait(b):
        pltpu.make_async_copy(bufms[b], acc.at[idxss[b]], semss[b]).wait()

    for q in range(4):
        idx_issue(q, q)
    for b in range(2):
        idx_wait(b, b)
        gather(b, b)
    plsc.subcore_barrier()

    def loop_body(jj, _):
        for t in range(4):
            b = t % 2
            q = t
            j = jj * 4 + t
            gather_wait(q, b)

            @pl.when(j >= 2)
            def _():
                scatter_wait(b)

            _edge_chunk(bufus[b], bufvs[b], bufms[b])
            for o in (0, 16, CHUNK - L):
                idxss[b][pl.ds(o, L)] = idxcs[q][pl.ds(o, L)]
            scatter(b)

            @pl.when(j + 2 < NCHUNK)
            def _():
                idx_wait(j + 2, (t + 2) % 4)
                gather((t + 2) % 4, b)

            @pl.when(j + 4 < NCHUNK)
            def _():
                idx_issue(j + 4, q)
        return 0

    lax.fori_loop(0, NCHUNK // 4, loop_body, 0)
    for b in range(2):
        scatter_wait(b)
    plsc.subcore_barrier()

    pltpu.sync_copy(acc.at[pl.ds(s * RPT, RPT)], out_hbm.at[pl.ds(s * RPT, RPT)])


@functools.lru_cache(maxsize=None)
def _sc_message_kernel():
    @functools.partial(
        pl.kernel,
        out_type=(
            jax.ShapeDtypeStruct((NPAD, DIM), jnp.float32),
            jax.ShapeDtypeStruct((NPAD, DIM), jnp.float32),
        ),
        mesh=_mesh(),
        scratch_types=[
            [pltpu.VMEM((CHUNK,), jnp.int32)] * 4,
            [pltpu.VMEM((CHUNK,), jnp.int32)] * 4,
            [pltpu.VMEM((CHUNK,), jnp.int32)] * 2,
            [pltpu.VMEM((CHUNK, DIM), jnp.float32)] * 2,
            [pltpu.VMEM((CHUNK, DIM), jnp.float32)] * 2,
            [pltpu.VMEM((CHUNK, DIM), jnp.float32)] * 2,
            pltpu.VMEM_SHARED((NPAD, DIM), jnp.float32),
            [pltpu.SemaphoreType.DMA] * 2,
            [pltpu.SemaphoreType.DMA] * 2,
            [pltpu.SemaphoreType.DMA] * 2,
            [pltpu.SemaphoreType.DMA] * 4,
            [pltpu.SemaphoreType.DMA] * 4,
        ],
    )
    def _sc_message_impl(row, col, u0, v0, u1, v1, s0_out, s1_out,
                         idxrs, idxcs, idxss, bufus, bufvs, bufms, acc,
                         semus, semvs, semss, semirs, semics):
        c = lax.axis_index("c")
        s = lax.axis_index("s")

        @pl.when(c == 0)
        def _():
            _msg_half(row, col, u0, v0, s0_out, s, idxrs, idxcs, idxss,
                      bufus, bufvs, bufms, acc, semus, semvs, semss,
                      semirs, semics)

        @pl.when(c == 1)
        def _():
            _msg_half(row, col, u1, v1, s1_out, s, idxrs, idxcs, idxss,
                      bufus, bufvs, bufms, acc, semus, semvs, semss,
                      semirs, semics)

    return _sc_message_impl


def _sc_message(row, col, u0, v0, u1, v1):
    return _sc_message_kernel()(row, col, u0, v0, u1, v1)


def _cnt_half(col, out_hbm, s, wid, idxc, ones_v, zbuf, acc):
    _zero_fill(zbuf, ZROWS)
    for i in range(RPT // ZROWS):
        pltpu.sync_copy(zbuf, acc.at[pl.ds(s * RPT + i * ZROWS, ZROWS)])

    one = jnp.full((L,), 1.0, jnp.float32)

    def fill_ones(r, _):
        for k in range(DIM // L):
            ones_v[r, pl.ds(k * L, L)] = one
        return 0

    lax.fori_loop(0, CHUNK, fill_ones, 0)
    plsc.subcore_barrier()

    def chunk_body(j, _):
        base = wid * EPC + j * CHUNK
        pltpu.sync_copy(col.at[pl.ds(base, CHUNK)], idxc)
        pltpu.sync_copy(ones_v, acc.at[idxc], add=True)
        return 0

    lax.fori_loop(0, NCHUNK_CNT, chunk_body, 0)
    plsc.subcore_barrier()
    pltpu.sync_copy(acc.at[pl.ds(s * RPT, RPT)], out_hbm.at[pl.ds(s * RPT, RPT)])


@functools.lru_cache(maxsize=None)
def _sc_count_kernel():
    @functools.partial(
        pl.kernel,
        out_type=(
            jax.ShapeDtypeStruct((NPAD, DIM), jnp.float32),
            jax.ShapeDtypeStruct((NPAD, DIM), jnp.float32),
        ),
        mesh=_mesh(),
        scratch_types=[
            pltpu.VMEM((CHUNK,), jnp.int32),
            pltpu.VMEM((CHUNK, DIM), jnp.float32),
            pltpu.VMEM((ZROWS, DIM), jnp.float32),
            pltpu.VMEM_SHARED((NPAD, DIM), jnp.float32),
        ],
    )
    def _sc_count_impl(col, c0_out, c1_out, idxc, ones_v, zbuf, acc):
        c = lax.axis_index("c")
        s = lax.axis_index("s")

        @pl.when(c == 0)
        def _():
            _cnt_half(col, c0_out, s, s, idxc, ones_v, zbuf, acc)

        @pl.when(c == 1)
        def _():
            _cnt_half(col, c1_out, s, NS + s, idxc, ones_v, zbuf, acc)

    return _sc_count_impl


def _sc_count(col):
    return _sc_count_kernel()(col)


BN = 1000
GRID = N // BN


def _ln(x, g, b):
    mu = jnp.mean(x, axis=-1, keepdims=True)
    var = jnp.mean(x * x, axis=-1, keepdims=True) - mu * mu
    return (x - mu) * lax.rsqrt(var + 1e-5) * g + b


def _tc_init_body(mv_ref, sc_ref, pos_ref, wa_ref, wb_ref, wc_ref, bm_ref,
                  u0_ref, v0_ref, u1_ref, v1_ref):
    pw = jnp.dot(pos_ref[...], wc_ref[...], preferred_element_type=jnp.float32)
    bm = bm_ref[...]
    for h_ref, u_ref, v_ref in ((mv_ref, u0_ref, v0_ref), (sc_ref, u1_ref, v1_ref)):
        h = h_ref[...]
        u_ref[...] = jnp.dot(h, wa_ref[...], preferred_element_type=jnp.float32) + pw + bm
        v_ref[...] = jnp.dot(h, wb_ref[...], preferred_element_type=jnp.float32) - pw


def _tc_upd_body(make_uv, mv_ref, sc_ref, s0_ref, s1_ref, c0_ref, c1_ref,
                 pos_ref, gm_ref, bem_ref, wu_ref, bu_ref, gu_ref, beu_ref,
                 wa_ref, wb_ref, wc_ref, bm_ref, *out_refs):
    cnt = c0_ref[...][:, :1] + c1_ref[...][:, :1]
    inv = 1.0 / jnp.maximum(cnt, 1.0)
    present = jnp.where(cnt > 0, 1.0, 0.0)
    gm = gm_ref[...]
    bem = bem_ref[...]
    wu = wu_ref[...]
    bu = bu_ref[...]
    gu = gu_ref[...]
    beu = beu_ref[...]
    if make_uv:
        mvo_ref, sco_ref, u0_ref, v0_ref, u1_ref, v1_ref = out_refs
        pw = jnp.dot(pos_ref[...], wc_ref[...], preferred_element_type=jnp.float32)
        pairs = ((mv_ref, s0_ref, mvo_ref, u0_ref, v0_ref),
                 (sc_ref, s1_ref, sco_ref, u1_ref, v1_ref))
    else:
        mvo_ref, sco_ref = out_refs
        pairs = ((mv_ref, s0_ref, mvo_ref, None, None),
                 (sc_ref, s1_ref, sco_ref, None, None))
    for h_ref, s_ref, ho_ref, u_ref, v_ref in pairs:
        h = h_ref[...]
        agg = s_ref[...] * inv * gm + bem * present
        upd = jnp.dot(h, wu[:DIM], preferred_element_type=jnp.float32)
        upd = upd + jnp.dot(agg, wu[DIM:], preferred_element_type=jnp.float32) + bu
        hn = h + _ln(upd, gu, beu)
        ho_ref[...] = hn
        if make_uv:
            u_ref[...] = jnp.dot(hn, wa_ref[...], preferred_element_type=jnp.float32) + pw + bm_ref[...]
            v_ref[...] = jnp.dot(hn, wb_ref[...], preferred_element_type=jnp.float32) - pw


def _row_spec(rows, cols):
    return pl.BlockSpec((rows, cols), lambda i: (i, 0))


def _full_spec(rows, cols):
    return pl.BlockSpec((rows, cols), lambda i: (0, 0))


_NODE = _row_spec(BN, DIM)
_OUT_NODE = jax.ShapeDtypeStruct((N, DIM), jnp.float32)


def _tc_init(mv, sc, pos, wa, wb, wc, bm):
    return pl.pallas_call(
        _tc_init_body,
        grid=(GRID,),
        in_specs=[_NODE, _NODE, _row_spec(BN, MV16), _full_spec(DIM, DIM),
                  _full_spec(DIM, DIM), _full_spec(MV16, DIM), _full_spec(1, DIM)],
        out_specs=[_NODE] * 4,
        out_shape=[_OUT_NODE] * 4,
    )(mv, sc, pos, wa, wb, wc, bm)


def _tc_update(mv, sc, s0, s1, c0, c1, pos, gm, bem, wu, bu, gu, beu,
               wa, wb, wc, bm, make_uv):
    n_out = 6 if make_uv else 2
    return pl.pallas_call(
        functools.partial(_tc_upd_body, make_uv),
        grid=(GRID,),
        in_specs=[_NODE, _NODE, _NODE, _NODE, _row_spec(BN, DIM), _row_spec(BN, DIM),
                  _row_spec(BN, MV16), _full_spec(1, DIM), _full_spec(1, DIM),
                  _full_spec(2 * DIM, DIM), _full_spec(1, DIM), _full_spec(1, DIM),
                  _full_spec(1, DIM), _full_spec(DIM, DIM), _full_spec(DIM, DIM),
                  _full_spec(MV16, DIM), _full_spec(1, DIM)],
        out_specs=[_NODE] * n_out,
        out_shape=[_OUT_NODE] * n_out,
    )(mv, sc, s0, s1, c0, c1, pos, gm, bem, wu, bu, gu, beu, wa, wb, wc, bm)


def kernel(mv, sc, pos, edge_index, W_msg, b_msg, g_msg, be_msg,
           W_upd, b_upd, g_upd, be_upd):
    row = edge_index[0]
    col = edge_index[1]
    c0, c1 = _sc_count(col)

    def r1(x):
        return x.reshape(1, DIM)

    wa = [W_msg[t, :DIM] for t in range(STEPS)]
    wb = [W_msg[t, DIM:2 * DIM] for t in range(STEPS)]
    wc = [W_msg[t, 2 * DIM:] for t in range(STEPS)]

    u0, v0, u1, v1 = _tc_init(mv, sc, pos, wa[0], wb[0], wc[0], r1(b_msg[0]))
    for t in range(STEPS):
        s0, s1 = _sc_message(row, col, u0, v0, u1, v1)
        tn = min(t + 1, STEPS - 1)
        outs = _tc_update(
            mv, sc, s0, s1, c0, c1, pos, r1(g_msg[t]), r1(be_msg[t]),
            W_upd[t], r1(b_upd[t]), r1(g_upd[t]), r1(be_upd[t]),
            wa[tn], wb[tn], wc[tn], r1(b_msg[tn]), make_uv=(t + 1 < STEPS))
        if t + 1 < STEPS:
            mv, sc, u0, v0, u1, v1 = outs
        else:
            mv, sc = outs
    return mv, sc
